# Initial kernel scaffold; baseline (speedup 1.0000x reference)
import jax
import jax.numpy as jnp
from jax.experimental import pallas as pl


def kernel(x, edge_index, edge_attr, W_l0, b_l0, W_r0, b_r0, W_e0, att0, bias0, W_l1, b_l1, W_r1, b_r1, W_e1, att1, bias1, W_lin, b_lin):
    raise NotImplementedError("write your pallas kernel here")


# speedup vs baseline: 4.5872x; 4.5872x over previous
import functools

import jax
import jax.numpy as jnp
from jax import lax
from jax.experimental import pallas as pl
from jax.experimental.pallas import tpu as pltpu
from jax.experimental.pallas import tpu_sc as plsc

N_NODES = 10000
NP = 10240
F = 128
HEADS = 2
CH = 64
E_RAW = 320000
ET = E_RAW + N_NODES
NC, NS, LANES = 2, 16, 16
NW = NC * NS
K = 128
EPW = 10368
ETP = NW * EPW
NCHUNKS = EPW // K
STRIPE = NP // NS

_mesh = plsc.VectorSubcoreMesh(
    core_axis_name="c", subcore_axis_name="s", num_cores=NC, num_subcores=NS)


def _mm2_body(x_ref, wl_ref, bl_ref, wr_ref, br_ref, ol_ref, or_ref):
    x = x_ref[...]
    ol_ref[...] = jnp.dot(x, wl_ref[...], preferred_element_type=jnp.float32) + bl_ref[...]
    or_ref[...] = jnp.dot(x, wr_ref[...], preferred_element_type=jnp.float32) + br_ref[...]


def _mm2(x, W_l, b_l, W_r, b_r):
    n = x.shape[0]
    blk = 2048
    return pl.pallas_call(
        _mm2_body,
        grid=(n // blk,),
        in_specs=[
            pl.BlockSpec((blk, F), lambda i: (i, 0)),
            pl.BlockSpec((F, F), lambda i: (0, 0)),
            pl.BlockSpec((F,), lambda i: (0,)),
            pl.BlockSpec((F, F), lambda i: (0, 0)),
            pl.BlockSpec((F,), lambda i: (0,)),
        ],
        out_specs=[
            pl.BlockSpec((blk, F), lambda i: (i, 0)),
            pl.BlockSpec((blk, F), lambda i: (i, 0)),
        ],
        out_shape=[
            jax.ShapeDtypeStruct((n, F), jnp.float32),
            jax.ShapeDtypeStruct((n, F), jnp.float32),
        ],
    )(x, W_l, b_l, W_r, b_r)


def _recip_body(d_ref, o_ref):
    o_ref[...] = 1.0 / (d_ref[0] + d_ref[1] + 1e-16)


def _recip(denoms):
    return pl.pallas_call(
        _recip_body,
        out_shape=jax.ShapeDtypeStruct((NP, 2), jnp.float32),
    )(denoms)


def _merge_relu_body(p_ref, b_ref, o_ref):
    o_ref[...] = jnp.maximum(p_ref[0] + p_ref[1] + b_ref[...], 0.0)


def _merge_relu(parts, bias):
    blk = 2048
    return pl.pallas_call(
        _merge_relu_body,
        grid=(NP // blk,),
        in_specs=[
            pl.BlockSpec((2, blk, F), lambda i: (0, i, 0)),
            pl.BlockSpec((F,), lambda i: (0,)),
        ],
        out_specs=pl.BlockSpec((blk, F), lambda i: (i, 0)),
        out_shape=jax.ShapeDtypeStruct((NP, F), jnp.float32),
    )(parts, bias)


def _merge_mm_final_body(p_ref, b_ref, w_ref, bw_ref, o_ref):
    h = jnp.maximum(p_ref[0] + p_ref[1] + b_ref[...], 0.0)
    y = jnp.dot(h, w_ref[...], preferred_element_type=jnp.float32) + bw_ref[...]
    o_ref[...] = jnp.maximum(y, 0.0)


def _merge_mm_final(parts, bias, W_lin, b_lin):
    blk = 2048
    return pl.pallas_call(
        _merge_mm_final_body,
        grid=(NP // blk,),
        in_specs=[
            pl.BlockSpec((2, blk, F), lambda i: (0, i, 0)),
            pl.BlockSpec((F,), lambda i: (0,)),
            pl.BlockSpec((F, F), lambda i: (0, 0)),
            pl.BlockSpec((F,), lambda i: (0,)),
        ],
        out_specs=pl.BlockSpec((blk, F), lambda i: (i, 0)),
        out_shape=jax.ShapeDtypeStruct((NP, F), jnp.float32),
    )(parts, bias, W_lin, b_lin)


@functools.partial(
    pl.kernel,
    out_type=(
        jax.ShapeDtypeStruct((ETP,), jnp.float32),
        jax.ShapeDtypeStruct((ETP,), jnp.float32),
        jax.ShapeDtypeStruct((NC, NP, 2), jnp.float32),
    ),
    mesh=_mesh,
    scratch_types=[
        pltpu.VMEM((K,), jnp.int32),
        pltpu.VMEM((K,), jnp.int32),
        pltpu.VMEM((K,), jnp.float32),
        pltpu.VMEM((K, F), jnp.float32),
        pltpu.VMEM((K, F), jnp.float32),
        pltpu.VMEM((K,), jnp.float32),
        pltpu.VMEM((K,), jnp.float32),
        pltpu.VMEM((K, 2), jnp.float32),
        pltpu.VMEM((F,), jnp.float32),
        pltpu.VMEM((F,), jnp.float32),
        pltpu.VMEM_SHARED((NP, 2), jnp.float32),
        pltpu.SemaphoreType.DMA,
    ],
    compiler_params=pltpu.CompilerParams(needs_layout_passes=False, use_tc_tiling_on_sc=False),
)
def _sc_pass_a(xl_hbm, xr_hbm, src_hbm, dst_hbm, ea_hbm, w_hbm, att_hbm,
               zeros2_hbm, ex0_hbm, ex1_hbm, den_hbm,
               src_v, dst_v, ea_v, rows_l, rows_r, ex0_v, ex1_v, dctr,
               w_v, att_v, den_sp, sem):
    cid = lax.axis_index("c")
    sid = lax.axis_index("s")
    wid = sid * NC + cid
    pltpu.sync_copy(w_hbm, w_v)
    pltpu.sync_copy(att_hbm, att_v)

    @pl.when(sid == 0)
    def _():
        pltpu.sync_copy(zeros2_hbm, den_sp)

    plsc.subcore_barrier()

    iota = lax.iota(jnp.int32, LANES)
    zeros16 = jnp.zeros((LANES,), jnp.int32)

    def chunk_body(ch, _):
        base = wid * EPW + ch * K
        pltpu.sync_copy(src_hbm.at[pl.ds(base, K)], src_v)
        pltpu.sync_copy(dst_hbm.at[pl.ds(base, K)], dst_v)
        pltpu.sync_copy(ea_hbm.at[pl.ds(base, K)], ea_v)
        pltpu.async_copy(xl_hbm.at[src_v], rows_l, sem).wait()
        pltpu.async_copy(xr_hbm.at[dst_v], rows_r, sem).wait()

        def group_body(g, _):
            row16 = g * LANES + iota
            ea16 = ea_v[pl.ds(g * LANES, LANES)]

            def blk_body(cb, accs):
                acc0, acc1 = accs
                cbase = cb * LANES
                wb0 = w_v[pl.ds(cbase, LANES)]
                wb1 = w_v[pl.ds(cbase + CH, LANES)]
                ab0 = att_v[pl.ds(cbase, LANES)]
                ab1 = att_v[pl.ds(cbase + CH, LANES)]
                for j in range(LANES):
                    cv = zeros16 + (cbase + j)
                    l0 = plsc.load_gather(rows_l, [row16, cv])
                    r0 = plsc.load_gather(rows_r, [row16, cv])
                    l1 = plsc.load_gather(rows_l, [row16, cv + CH])
                    r1 = plsc.load_gather(rows_r, [row16, cv + CH])
                    m0 = l0 + r0 + ea16 * wb0[j]
                    m0 = jnp.maximum(m0, 0.2 * m0)
                    m1 = l1 + r1 + ea16 * wb1[j]
                    m1 = jnp.maximum(m1, 0.2 * m1)
                    acc0 = acc0 + m0 * ab0[j]
                    acc1 = acc1 + m1 * ab1[j]
                return (acc0, acc1)

            acc0, acc1 = lax.fori_loop(
                0, CH // LANES, blk_body,
                (jnp.zeros((LANES,), jnp.float32),
                 jnp.zeros((LANES,), jnp.float32)))
            eid = base + g * LANES + iota
            mask = eid < ET
            e0 = jnp.where(mask, jnp.exp(acc0), 0.0)
            e1 = jnp.where(mask, jnp.exp(acc1), 0.0)
            ex0_v[pl.ds(g * LANES, LANES)] = e0
            ex1_v[pl.ds(g * LANES, LANES)] = e1
            plsc.store_scatter(dctr, [row16, zeros16], e0)
            plsc.store_scatter(dctr, [row16, zeros16 + 1], e1)
            return 0

        lax.fori_loop(0, K // LANES, group_body, 0)
        pltpu.sync_copy(ex0_v, ex0_hbm.at[pl.ds(base, K)])
        pltpu.sync_copy(ex1_v, ex1_hbm.at[pl.ds(base, K)])
        pltpu.sync_copy(dctr, den_sp.at[dst_v], add=True)
        return 0

    lax.fori_loop(0, NCHUNKS, chunk_body, 0)
    plsc.subcore_barrier()
    pltpu.sync_copy(den_sp.at[pl.ds(sid * STRIPE, STRIPE)],
                    den_hbm.at[cid, pl.ds(sid * STRIPE, STRIPE)])


FS = 8
NSW = F // FS

@functools.partial(
    pl.kernel,
    out_type=jax.ShapeDtypeStruct((NC, NP, F), jnp.float32),
    mesh=_mesh,
    scratch_types=[
        pltpu.VMEM((K,), jnp.int32),
        pltpu.VMEM((K,), jnp.int32),
        pltpu.VMEM((K,), jnp.float32),
        pltpu.VMEM((K,), jnp.float32),
        pltpu.VMEM((K, FS), jnp.float32),
        pltpu.VMEM((K, FS), jnp.float32),
        pltpu.VMEM((NP, 2), jnp.float32),
        pltpu.VMEM_SHARED((NP, FS), jnp.float32),
        pltpu.SemaphoreType.DMA,
        pltpu.VMEM((K,), jnp.int32),
    ],
    compiler_params=pltpu.CompilerParams(needs_layout_passes=False, use_tc_tiling_on_sc=False),
)
def _sc_pass_b(xlcg_hbm, src_hbm, dst_hbm, ex0_hbm, ex1_hbm, recip_hbm,
               zeros8_hbm, out_hbm,
               src_v, dst_v, ex0_v, ex1_v, rows, ctr, recip_v, acc_sp, sem,
               idx_v):
    cid = lax.axis_index("c")
    sid = lax.axis_index("s")
    wid = sid * NC + cid
    pltpu.sync_copy(recip_hbm, recip_v)

    iota = lax.iota(jnp.int32, LANES)
    zeros16 = jnp.zeros((LANES,), jnp.int32)

    def sweep_body(cg, _):
        @pl.when(sid == 0)
        def _():
            pltpu.sync_copy(zeros8_hbm, acc_sp)

        plsc.subcore_barrier()

        def chunk_body(ch, _):
            base = wid * EPW + ch * K
            pltpu.sync_copy(src_hbm.at[pl.ds(base, K)], src_v)
            pltpu.sync_copy(dst_hbm.at[pl.ds(base, K)], dst_v)
            pltpu.sync_copy(ex0_hbm.at[pl.ds(base, K)], ex0_v)
            pltpu.sync_copy(ex1_hbm.at[pl.ds(base, K)], ex1_v)

            def idx_body(g, _):
                sl = pl.ds(g * LANES, LANES)
                idx_v[sl] = src_v[sl] + cg * NP
                return 0

            lax.fori_loop(0, K // LANES, idx_body, 0)
            pltpu.async_copy(xlcg_hbm.at[idx_v], rows, sem).wait()

            def group_body(g, _):
                row16 = g * LANES + iota
                d16 = dst_v[pl.ds(g * LANES, LANES)]
                a0 = ex0_v[pl.ds(g * LANES, LANES)] * plsc.load_gather(
                    recip_v, [d16, zeros16])
                a1 = ex1_v[pl.ds(g * LANES, LANES)] * plsc.load_gather(
                    recip_v, [d16, zeros16 + 1])
                a = jnp.where(cg < NSW // 2, a0, a1)
                for j in range(FS):
                    cv = zeros16 + j
                    l = plsc.load_gather(rows, [row16, cv])
                    plsc.store_scatter(ctr, [row16, cv], l * a)
                return 0

            lax.fori_loop(0, K // LANES, group_body, 0)
            pltpu.sync_copy(ctr, acc_sp.at[dst_v], add=True)
            return 0

        lax.fori_loop(0, NCHUNKS, chunk_body, 0)
        plsc.subcore_barrier()
        pltpu.sync_copy(
            acc_sp.at[pl.ds(sid * STRIPE, STRIPE)],
            out_hbm.at[cid, pl.ds(sid * STRIPE, STRIPE), pl.ds(cg * FS, FS)])
        plsc.subcore_barrier()
        return 0

    lax.fori_loop(0, NSW, sweep_body, 0)


_SKIP_B = False

def _gat_layer(x_np, src_p, dst_p, ea_p, W_l, b_l, W_r, b_r, W_e, att,
               zeros2, zeros8):
    x_l, x_r = _mm2(x_np, W_l, b_l, W_r, b_r)
    w_vec = W_e.reshape(F)
    att_vec = att.reshape(F)
    ex0, ex1, denoms = _sc_pass_a(
        x_l, x_r, src_p, dst_p, ea_p, w_vec, att_vec, zeros2)
    recip = _recip(denoms)
    if _SKIP_B:
        return jnp.broadcast_to(recip[None, :, :1], (NC, NP, F)) * 0.0 + ex0[0] + ex1[0]
    xlcg = x_l.reshape(NP, NSW, FS).transpose(1, 0, 2).reshape(NSW * NP, FS)
    parts = _sc_pass_b(xlcg, src_p, dst_p, ex0, ex1, recip, zeros8)
    return parts


def kernel(x, edge_index, edge_attr, W_l0, b_l0, W_r0, b_r0, W_e0, att0,
           bias0, W_l1, b_l1, W_r1, b_r1, W_e1, att1, bias1, W_lin, b_lin):
    x = x.astype(jnp.float32)
    edge_attr = edge_attr.astype(jnp.float32)

    loop = jnp.arange(N_NODES, dtype=edge_index.dtype)
    src = jnp.concatenate([edge_index[0], loop])
    dst = jnp.concatenate([edge_index[1], loop])
    mean_attr = jnp.mean(edge_attr)
    ea = jnp.concatenate(
        [edge_attr[:, 0], jnp.full((N_NODES,), mean_attr, jnp.float32)])
    pad = ETP - ET
    src_p = jnp.pad(src, (0, pad)).astype(jnp.int32)
    dst_p = jnp.pad(dst, (0, pad)).astype(jnp.int32)
    ea_p = jnp.pad(ea, (0, pad))

    x_np = jnp.pad(x, ((0, NP - N_NODES), (0, 0)))
    zeros2 = jnp.zeros((NP, 2), jnp.float32)
    zeros8 = jnp.zeros((NP, FS), jnp.float32)

    parts0 = _gat_layer(x_np, src_p, dst_p, ea_p, W_l0, b_l0, W_r0, b_r0,
                        W_e0, att0, zeros2, zeros8)
    h = _merge_relu(parts0, bias0)
    parts1 = _gat_layer(h, src_p, dst_p, ea_p, W_l1, b_l1, W_r1, b_r1,
                        W_e1, att1, zeros2, zeros8)
    y = _merge_mm_final(parts1, bias1, W_lin, b_lin)
    return y[:N_NODES]


# speedup vs baseline: 7.2382x; 1.5779x over previous
import functools

import jax
import jax.numpy as jnp
from jax import lax
from jax.experimental import pallas as pl
from jax.experimental.pallas import tpu as pltpu
from jax.experimental.pallas import tpu_sc as plsc

N_NODES = 10000
NP = 10240
F = 128
HEADS = 2
CH = 64
E_RAW = 320000
ET = E_RAW + N_NODES
NC, NS, LANES = 2, 16, 16
NW = NC * NS
K = 128
EPW = 10368
ETP = NW * EPW
NCHUNKS = EPW // K
STRIPE = NP // NS

_mesh = plsc.VectorSubcoreMesh(
    core_axis_name="c", subcore_axis_name="s", num_cores=NC, num_subcores=NS)


def _mm2_body(x_ref, wl_ref, bl_ref, wr_ref, br_ref, ol_ref, or_ref):
    x = x_ref[...]
    ol_ref[...] = jnp.dot(x, wl_ref[...], preferred_element_type=jnp.float32) + bl_ref[...]
    or_ref[...] = jnp.dot(x, wr_ref[...], preferred_element_type=jnp.float32) + br_ref[...]


def _mm2(x, W_l, b_l, W_r, b_r):
    n = x.shape[0]
    blk = 2048
    return pl.pallas_call(
        _mm2_body,
        grid=(n // blk,),
        in_specs=[
            pl.BlockSpec((blk, F), lambda i: (i, 0)),
            pl.BlockSpec((F, F), lambda i: (0, 0)),
            pl.BlockSpec((F,), lambda i: (0,)),
            pl.BlockSpec((F, F), lambda i: (0, 0)),
            pl.BlockSpec((F,), lambda i: (0,)),
        ],
        out_specs=[
            pl.BlockSpec((blk, F), lambda i: (i, 0)),
            pl.BlockSpec((blk, F), lambda i: (i, 0)),
        ],
        out_shape=[
            jax.ShapeDtypeStruct((n, F), jnp.float32),
            jax.ShapeDtypeStruct((n, F), jnp.float32),
        ],
    )(x, W_l, b_l, W_r, b_r)


def _recip_body(d_ref, o_ref):
    o_ref[...] = 1.0 / (d_ref[0] + d_ref[1] + 1e-16)


def _recip(denoms):
    return pl.pallas_call(
        _recip_body,
        out_shape=jax.ShapeDtypeStruct((NP, 2), jnp.float32),
    )(denoms)


def _merge_relu_body(p_ref, b_ref, o_ref):
    o_ref[...] = jnp.maximum(p_ref[0] + p_ref[1] + b_ref[...], 0.0)


def _merge_relu(parts, bias):
    blk = 2048
    return pl.pallas_call(
        _merge_relu_body,
        grid=(NP // blk,),
        in_specs=[
            pl.BlockSpec((2, blk, F), lambda i: (0, i, 0)),
            pl.BlockSpec((F,), lambda i: (0,)),
        ],
        out_specs=pl.BlockSpec((blk, F), lambda i: (i, 0)),
        out_shape=jax.ShapeDtypeStruct((NP, F), jnp.float32),
    )(parts, bias)


def _merge_mm_final_body(p_ref, b_ref, w_ref, bw_ref, o_ref):
    h = jnp.maximum(p_ref[0] + p_ref[1] + b_ref[...], 0.0)
    y = jnp.dot(h, w_ref[...], preferred_element_type=jnp.float32) + bw_ref[...]
    o_ref[...] = jnp.maximum(y, 0.0)


def _merge_mm_final(parts, bias, W_lin, b_lin):
    blk = 2048
    return pl.pallas_call(
        _merge_mm_final_body,
        grid=(NP // blk,),
        in_specs=[
            pl.BlockSpec((2, blk, F), lambda i: (0, i, 0)),
            pl.BlockSpec((F,), lambda i: (0,)),
            pl.BlockSpec((F, F), lambda i: (0, 0)),
            pl.BlockSpec((F,), lambda i: (0,)),
        ],
        out_specs=pl.BlockSpec((blk, F), lambda i: (i, 0)),
        out_shape=jax.ShapeDtypeStruct((NP, F), jnp.float32),
    )(parts, bias, W_lin, b_lin)


@functools.partial(
    pl.kernel,
    out_type=(
        jax.ShapeDtypeStruct((ETP,), jnp.float32),
        jax.ShapeDtypeStruct((ETP,), jnp.float32),
        jax.ShapeDtypeStruct((NC, NP, 2), jnp.float32),
    ),
    mesh=_mesh,
    scratch_types=[
        pltpu.VMEM((K,), jnp.int32),
        pltpu.VMEM((K,), jnp.int32),
        pltpu.VMEM((K,), jnp.float32),
        pltpu.VMEM((K, F), jnp.float32),
        pltpu.VMEM((K, F), jnp.float32),
        pltpu.VMEM((K,), jnp.float32),
        pltpu.VMEM((K,), jnp.float32),
        pltpu.VMEM((K, 2), jnp.float32),
        pltpu.VMEM((F,), jnp.float32),
        pltpu.VMEM((F,), jnp.float32),
        pltpu.VMEM_SHARED((NP, 2), jnp.float32),
        pltpu.SemaphoreType.DMA,
    ],
    compiler_params=pltpu.CompilerParams(needs_layout_passes=False, use_tc_tiling_on_sc=False),
)
def _sc_pass_a(xl_hbm, xr_hbm, src_hbm, dst_hbm, ea_hbm, w_hbm, att_hbm,
               zeros2_hbm, ex0_hbm, ex1_hbm, den_hbm,
               src_v, dst_v, ea_v, rows_l, rows_r, ex0_v, ex1_v, dctr,
               w_v, att_v, den_sp, sem):
    cid = lax.axis_index("c")
    sid = lax.axis_index("s")
    wid = sid * NC + cid
    pltpu.sync_copy(w_hbm, w_v)
    pltpu.sync_copy(att_hbm, att_v)

    @pl.when(sid == 0)
    def _():
        pltpu.sync_copy(zeros2_hbm, den_sp)

    plsc.subcore_barrier()

    iota = lax.iota(jnp.int32, LANES)
    zeros16 = jnp.zeros((LANES,), jnp.int32)

    def chunk_body(ch, _):
        base = wid * EPW + ch * K
        pltpu.sync_copy(src_hbm.at[pl.ds(base, K)], src_v)
        pltpu.sync_copy(dst_hbm.at[pl.ds(base, K)], dst_v)
        pltpu.sync_copy(ea_hbm.at[pl.ds(base, K)], ea_v)
        pltpu.async_copy(xl_hbm.at[src_v], rows_l, sem).wait()
        pltpu.async_copy(xr_hbm.at[dst_v], rows_r, sem).wait()

        def group_body(g, _):
            row16 = g * LANES + iota
            ea16 = ea_v[pl.ds(g * LANES, LANES)]

            def blk_body(cb, accs):
                acc0, acc1 = accs
                cbase = cb * LANES
                wb0 = w_v[pl.ds(cbase, LANES)]
                wb1 = w_v[pl.ds(cbase + CH, LANES)]
                ab0 = att_v[pl.ds(cbase, LANES)]
                ab1 = att_v[pl.ds(cbase + CH, LANES)]
                for j in range(LANES):
                    cv = zeros16 + (cbase + j)
                    l0 = plsc.load_gather(rows_l, [row16, cv])
                    r0 = plsc.load_gather(rows_r, [row16, cv])
                    l1 = plsc.load_gather(rows_l, [row16, cv + CH])
                    r1 = plsc.load_gather(rows_r, [row16, cv + CH])
                    m0 = l0 + r0 + ea16 * wb0[j]
                    m0 = jnp.maximum(m0, 0.2 * m0)
                    m1 = l1 + r1 + ea16 * wb1[j]
                    m1 = jnp.maximum(m1, 0.2 * m1)
                    acc0 = acc0 + m0 * ab0[j]
                    acc1 = acc1 + m1 * ab1[j]
                return (acc0, acc1)

            acc0, acc1 = lax.fori_loop(
                0, CH // LANES, blk_body,
                (jnp.zeros((LANES,), jnp.float32),
                 jnp.zeros((LANES,), jnp.float32)))
            eid = base + g * LANES + iota
            mask = eid < ET
            e0 = jnp.where(mask, jnp.exp(acc0), 0.0)
            e1 = jnp.where(mask, jnp.exp(acc1), 0.0)
            ex0_v[pl.ds(g * LANES, LANES)] = e0
            ex1_v[pl.ds(g * LANES, LANES)] = e1
            plsc.store_scatter(dctr, [row16, zeros16], e0)
            plsc.store_scatter(dctr, [row16, zeros16 + 1], e1)
            return 0

        lax.fori_loop(0, K // LANES, group_body, 0)
        pltpu.sync_copy(ex0_v, ex0_hbm.at[pl.ds(base, K)])
        pltpu.sync_copy(ex1_v, ex1_hbm.at[pl.ds(base, K)])
        pltpu.sync_copy(dctr, den_sp.at[dst_v], add=True)
        return 0

    lax.fori_loop(0, NCHUNKS, chunk_body, 0)
    plsc.subcore_barrier()
    pltpu.sync_copy(den_sp.at[pl.ds(sid * STRIPE, STRIPE)],
                    den_hbm.at[cid, pl.ds(sid * STRIPE, STRIPE)])


FS = 8
NSW = F // FS

@functools.partial(
    pl.kernel,
    out_type=jax.ShapeDtypeStruct((NC, NP, F), jnp.float32),
    mesh=_mesh,
    scratch_types=[
        pltpu.VMEM((EPW,), jnp.int32),
        pltpu.VMEM((EPW,), jnp.int32),
        pltpu.VMEM((EPW,), jnp.float32),
        pltpu.VMEM((EPW,), jnp.float32),
        pltpu.VMEM((K, FS), jnp.float32),
        pltpu.VMEM((K, FS), jnp.float32),
        pltpu.VMEM((NP, 2), jnp.float32),
        pltpu.VMEM_SHARED((NP, FS), jnp.float32),
        pltpu.SemaphoreType.DMA,
        pltpu.VMEM((K,), jnp.int32),
    ],
    compiler_params=pltpu.CompilerParams(needs_layout_passes=False, use_tc_tiling_on_sc=False),
)
def _sc_pass_b(xlcg_hbm, src_hbm, dst_hbm, ex0_hbm, ex1_hbm, recip_hbm,
               zeros8_hbm, out_hbm,
               src_v, dst_v, ex0_v, ex1_v, rows, ctr, recip_v, acc_sp, sem,
               idx_v):
    cid = lax.axis_index("c")
    sid = lax.axis_index("s")
    wid = sid * NC + cid
    pltpu.sync_copy(recip_hbm, recip_v)
    wbase = wid * EPW
    pltpu.sync_copy(src_hbm.at[pl.ds(wbase, EPW)], src_v)
    pltpu.sync_copy(dst_hbm.at[pl.ds(wbase, EPW)], dst_v)
    pltpu.sync_copy(ex0_hbm.at[pl.ds(wbase, EPW)], ex0_v)
    pltpu.sync_copy(ex1_hbm.at[pl.ds(wbase, EPW)], ex1_v)

    iota = lax.iota(jnp.int32, LANES)
    zeros16 = jnp.zeros((LANES,), jnp.int32)

    def sweep_body(cg, _):
        @pl.when(sid == 0)
        def _():
            pltpu.sync_copy(zeros8_hbm, acc_sp)

        plsc.subcore_barrier()

        def chunk_body(ch, _):
            eoff = ch * K

            def idx_body(g, _):
                sl = pl.ds(eoff + g * LANES, LANES)
                idx_v[pl.ds(g * LANES, LANES)] = src_v[sl] + cg * NP
                return 0

            lax.fori_loop(0, K // LANES, idx_body, 0)
            pltpu.async_copy(xlcg_hbm.at[idx_v], rows, sem).wait()

            def group_body(g, _):
                row16 = g * LANES + iota
                sl = pl.ds(eoff + g * LANES, LANES)
                d16 = dst_v[sl]
                a0 = ex0_v[sl] * plsc.load_gather(recip_v, [d16, zeros16])
                a1 = ex1_v[sl] * plsc.load_gather(recip_v, [d16, zeros16 + 1])
                a = jnp.where(cg < NSW // 2, a0, a1)
                for j in range(FS):
                    cv = zeros16 + j
                    l = plsc.load_gather(rows, [row16, cv])
                    plsc.store_scatter(ctr, [row16, cv], l * a)
                return 0

            lax.fori_loop(0, K // LANES, group_body, 0)
            pltpu.sync_copy(ctr, acc_sp.at[dst_v.at[pl.ds(eoff, K)]], add=True)
            return 0

        lax.fori_loop(0, NCHUNKS, chunk_body, 0)
        plsc.subcore_barrier()
        pltpu.sync_copy(
            acc_sp.at[pl.ds(sid * STRIPE, STRIPE)],
            out_hbm.at[cid, pl.ds(sid * STRIPE, STRIPE), pl.ds(cg * FS, FS)])
        plsc.subcore_barrier()
        return 0

    lax.fori_loop(0, NSW, sweep_body, 0)


_SKIP_B = False

def _gat_layer(x_np, src_p, dst_p, ea_p, W_l, b_l, W_r, b_r, W_e, att,
               zeros2, zeros8):
    x_l, x_r = _mm2(x_np, W_l, b_l, W_r, b_r)
    w_vec = W_e.reshape(F)
    att_vec = att.reshape(F)
    ex0, ex1, denoms = _sc_pass_a(
        x_l, x_r, src_p, dst_p, ea_p, w_vec, att_vec, zeros2)
    recip = _recip(denoms)
    if _SKIP_B:
        return jnp.broadcast_to(recip[None, :, :1], (NC, NP, F)) * 0.0 + ex0[0] + ex1[0]
    xlcg = x_l.reshape(NP, NSW, FS).transpose(1, 0, 2).reshape(NSW * NP, FS)
    parts = _sc_pass_b(xlcg, src_p, dst_p, ex0, ex1, recip, zeros8)
    return parts


def kernel(x, edge_index, edge_attr, W_l0, b_l0, W_r0, b_r0, W_e0, att0,
           bias0, W_l1, b_l1, W_r1, b_r1, W_e1, att1, bias1, W_lin, b_lin):
    x = x.astype(jnp.float32)
    edge_attr = edge_attr.astype(jnp.float32)

    loop = jnp.arange(N_NODES, dtype=edge_index.dtype)
    src = jnp.concatenate([edge_index[0], loop])
    dst = jnp.concatenate([edge_index[1], loop])
    mean_attr = jnp.mean(edge_attr)
    ea = jnp.concatenate(
        [edge_attr[:, 0], jnp.full((N_NODES,), mean_attr, jnp.float32)])
    pad = ETP - ET
    src_p = jnp.pad(src, (0, pad)).astype(jnp.int32)
    dst_p = jnp.pad(dst, (0, pad)).astype(jnp.int32)
    ea_p = jnp.pad(ea, (0, pad))

    x_np = jnp.pad(x, ((0, NP - N_NODES), (0, 0)))
    zeros2 = jnp.zeros((NP, 2), jnp.float32)
    zeros8 = jnp.zeros((NP, FS), jnp.float32)

    parts0 = _gat_layer(x_np, src_p, dst_p, ea_p, W_l0, b_l0, W_r0, b_r0,
                        W_e0, att0, zeros2, zeros8)
    h = _merge_relu(parts0, bias0)
    parts1 = _gat_layer(h, src_p, dst_p, ea_p, W_l1, b_l1, W_r1, b_r1,
                        W_e1, att1, zeros2, zeros8)
    y = _merge_mm_final(parts1, bias1, W_lin, b_lin)
    return y[:N_NODES]
